# baseline (speedup 1.0000x reference)
import functools

import jax
import jax.numpy as jnp
from jax import lax
from jax.experimental import pallas as pl
from jax.experimental.pallas import tpu as pltpu
from jax.experimental.pallas import tpu_sc as plsc

_NUM_EMB = 8192
_EMB_DIM = 256
_N_ROWS = 16384

_BI = 256
_NI = _N_ROWS // _BI

_NW = 32
_ROWS_PER_WORKER = _N_ROWS // _NW
_GATHER_CHUNK = 128
_N_CHUNKS = _ROWS_PER_WORKER // _GATHER_CHUNK


def _vq_body(x_ref, w_ref, g_ref, loss_ref, idx_ref, oh_ref):
    i = pl.program_id(0)

    @pl.when(i == 0)
    def _():
        loss_ref[0, 0] = 0.0

    d = lax.dot_general(
        x_ref[...], w_ref[...],
        dimension_numbers=(((1,), (1,)), ((), ())),
        preferred_element_type=jnp.float32,
    )
    nchunks = _NUM_EMB // 128
    vals = [d[:, j * 128:(j + 1) * 128] for j in range(nchunks)]
    chids = [jnp.full((_BI, 128), j, jnp.int32) for j in range(nchunks)]
    while len(vals) > 1:
        nv, nc = [], []
        for a in range(0, len(vals), 2):
            keep_l = vals[a] >= vals[a + 1]
            nv.append(jnp.where(keep_l, vals[a], vals[a + 1]))
            nc.append(jnp.where(keep_l, chids[a], chids[a + 1]))
        vals, chids = nv, nc
    v_lane, c_lane = vals[0], chids[0]

    m = jnp.max(v_lane, axis=1, keepdims=True)
    lane = lax.broadcasted_iota(jnp.int32, (_BI, 128), 1)
    qual = jnp.where(v_lane == m, c_lane * 128 + lane, _NUM_EMB)
    la = jnp.min(qual, axis=1, keepdims=True)
    idx_ref[...] = la
    cols = lax.broadcasted_iota(jnp.int32, d.shape, 1)
    oh_ref[...] = (cols == la).astype(jnp.float32)

    loss_ref[0, 0] += jnp.sum(x_ref[...] * g_ref[...])

    @pl.when(i == _NI - 1)
    def _():
        loss_ref[0, 0] = 1.0 - loss_ref[0, 0] / float(_N_ROWS)


_vq_call = pl.pallas_call(
    _vq_body,
    grid=(_NI,),
    in_specs=[
        pl.BlockSpec((_BI, _EMB_DIM), lambda i: (i, 0)),
        pl.BlockSpec((_NUM_EMB, _EMB_DIM), lambda i: (0, 0)),
        pl.BlockSpec((_BI, _EMB_DIM), lambda i: (i, 0)),
    ],
    out_specs=[
        pl.BlockSpec((1, 1), lambda i: (0, 0), memory_space=pltpu.SMEM),
        pl.BlockSpec((_BI, 1), lambda i: (i, 0)),
        pl.BlockSpec((_BI, _NUM_EMB), lambda i: (i, 0)),
    ],
    out_shape=[
        jax.ShapeDtypeStruct((1, 1), jnp.float32),
        jax.ShapeDtypeStruct((_N_ROWS, 1), jnp.int32),
        jax.ShapeDtypeStruct((_N_ROWS, _NUM_EMB), jnp.float32),
    ],
)


@functools.cache
def _make_sc_gather():
    @functools.partial(
        pl.kernel,
        mesh=plsc.VectorSubcoreMesh(core_axis_name="c", subcore_axis_name="s"),
        out_type=jax.ShapeDtypeStruct((_N_ROWS, _EMB_DIM), jnp.float32),
        scratch_types=[
            pltpu.VMEM((_GATHER_CHUNK,), jnp.int32),
            pltpu.VMEM((_GATHER_CHUNK, _EMB_DIM), jnp.float32),
            pltpu.SemaphoreType.DMA,
        ],
    )
    def _sc_gather(table_hbm, idx_hbm, out_hbm, idx_v, rows_v, sem):
        wid = lax.axis_index("s") * 2 + lax.axis_index("c")
        base = wid * _ROWS_PER_WORKER
        for c in range(_N_CHUNKS):
            off = base + c * _GATHER_CHUNK
            pltpu.sync_copy(idx_hbm.at[pl.ds(off, _GATHER_CHUNK)], idx_v)
            pltpu.async_copy(table_hbm.at[idx_v], rows_v, sem).wait()
            pltpu.sync_copy(rows_v, out_hbm.at[pl.ds(off, _GATHER_CHUNK)])

    return _sc_gather


def _l2norm(t):
    n = jnp.linalg.norm(t, axis=1, keepdims=True)
    return t / jnp.maximum(n, 1e-12)


def kernel(inputs, labels, W):
    flat = inputs.reshape(-1, _EMB_DIM)
    xn = _l2norm(flat)
    wn = _l2norm(W)
    labels_i32 = labels.astype(jnp.int32)

    sc_gather = _make_sc_gather()
    lab_rows = sc_gather(wn, labels_i32)

    loss2d, idx2d, encodings = _vq_call(xn, wn, lab_rows)

    quantized = sc_gather(W, idx2d.reshape(_N_ROWS))

    return (
        loss2d.reshape(()),
        quantized.reshape(inputs.shape),
        jnp.array(1),
        encodings,
        idx2d,
    )


# speedup vs baseline: 1.0585x; 1.0585x over previous
import functools

import jax
import jax.numpy as jnp
from jax import lax
from jax.experimental import pallas as pl
from jax.experimental.pallas import tpu as pltpu
from jax.experimental.pallas import tpu_sc as plsc

_NUM_EMB = 8192
_EMB_DIM = 256
_N_ROWS = 16384

_BI = 256
_NI = _N_ROWS // _BI

_NW = 32
_ROWS_PER_WORKER = _N_ROWS // _NW
_GATHER_CHUNK = 128
_N_CHUNKS = _ROWS_PER_WORKER // _GATHER_CHUNK


def _vq_body(x_ref, w_ref, g_ref, loss_ref, idx_ref, oh_ref):
    i = pl.program_id(0)

    @pl.when(i == 0)
    def _():
        loss_ref[0, 0] = 0.0

    xs = x_ref[...]
    xn = xs / jnp.maximum(
        jnp.sqrt(jnp.sum(xs * xs, axis=1, keepdims=True)), 1e-12)

    d = lax.dot_general(
        xn, w_ref[...],
        dimension_numbers=(((1,), (1,)), ((), ())),
        preferred_element_type=jnp.float32,
    )
    nchunks = _NUM_EMB // 128
    vals = [d[:, j * 128:(j + 1) * 128] for j in range(nchunks)]
    chids = [jnp.full((_BI, 128), j, jnp.int32) for j in range(nchunks)]
    while len(vals) > 1:
        nv, nc = [], []
        for a in range(0, len(vals), 2):
            keep_l = vals[a] >= vals[a + 1]
            nv.append(jnp.where(keep_l, vals[a], vals[a + 1]))
            nc.append(jnp.where(keep_l, chids[a], chids[a + 1]))
        vals, chids = nv, nc
    v_lane, c_lane = vals[0], chids[0]

    m = jnp.max(v_lane, axis=1, keepdims=True)
    lane = lax.broadcasted_iota(jnp.int32, (_BI, 128), 1)
    qual = jnp.where(v_lane == m, c_lane * 128 + lane, _NUM_EMB)
    la = jnp.min(qual, axis=1, keepdims=True)
    idx_ref[...] = la
    cols = lax.broadcasted_iota(jnp.int32, d.shape, 1)
    oh_ref[...] = (cols == la).astype(jnp.float32)

    gs = g_ref[...]
    gdot = jnp.sum(xn * gs, axis=1, keepdims=True)
    gnorm = jnp.maximum(jnp.sqrt(jnp.sum(gs * gs, axis=1, keepdims=True)),
                        1e-12)
    loss_ref[0, 0] += jnp.sum(gdot / gnorm)

    @pl.when(i == _NI - 1)
    def _():
        loss_ref[0, 0] = 1.0 - loss_ref[0, 0] / float(_N_ROWS)


_vq_call = pl.pallas_call(
    _vq_body,
    grid=(_NI,),
    in_specs=[
        pl.BlockSpec((_BI, _EMB_DIM), lambda i: (i, 0)),
        pl.BlockSpec((_NUM_EMB, _EMB_DIM), lambda i: (0, 0)),
        pl.BlockSpec((_BI, _EMB_DIM), lambda i: (i, 0)),
    ],
    out_specs=[
        pl.BlockSpec((1, 1), lambda i: (0, 0), memory_space=pltpu.SMEM),
        pl.BlockSpec((_BI, 1), lambda i: (i, 0)),
        pl.BlockSpec((_BI, _NUM_EMB), lambda i: (i, 0)),
    ],
    out_shape=[
        jax.ShapeDtypeStruct((1, 1), jnp.float32),
        jax.ShapeDtypeStruct((_N_ROWS, 1), jnp.int32),
        jax.ShapeDtypeStruct((_N_ROWS, _NUM_EMB), jnp.float32),
    ],
)


@functools.cache
def _make_sc_gather():
    @functools.partial(
        pl.kernel,
        mesh=plsc.VectorSubcoreMesh(core_axis_name="c", subcore_axis_name="s"),
        out_type=jax.ShapeDtypeStruct((_N_ROWS, _EMB_DIM), jnp.float32),
        scratch_types=[
            pltpu.VMEM((_GATHER_CHUNK,), jnp.int32),
            pltpu.VMEM((_GATHER_CHUNK, _EMB_DIM), jnp.float32),
            pltpu.SemaphoreType.DMA,
        ],
    )
    def _sc_gather(table_hbm, idx_hbm, out_hbm, idx_v, rows_v, sem):
        wid = lax.axis_index("s") * 2 + lax.axis_index("c")
        base = wid * _ROWS_PER_WORKER
        for c in range(_N_CHUNKS):
            off = base + c * _GATHER_CHUNK
            pltpu.sync_copy(idx_hbm.at[pl.ds(off, _GATHER_CHUNK)], idx_v)
            pltpu.async_copy(table_hbm.at[idx_v], rows_v, sem).wait()
            pltpu.sync_copy(rows_v, out_hbm.at[pl.ds(off, _GATHER_CHUNK)])

    return _sc_gather


def _l2norm(t):
    n = jnp.linalg.norm(t, axis=1, keepdims=True)
    return t / jnp.maximum(n, 1e-12)


def kernel(inputs, labels, W):
    flat = inputs.reshape(-1, _EMB_DIM)
    labels_i32 = labels.astype(jnp.int32)

    sc_gather = _make_sc_gather()
    lab_rows = sc_gather(W, labels_i32)

    wn = _l2norm(W)
    loss2d, idx2d, encodings = _vq_call(flat, wn, lab_rows)

    quantized = sc_gather(W, idx2d.reshape(_N_ROWS))

    return (
        loss2d.reshape(()),
        quantized.reshape(inputs.shape),
        jnp.array(1),
        encodings,
        idx2d,
    )


# speedup vs baseline: 1.1081x; 1.0469x over previous
import functools

import jax
import jax.numpy as jnp
from jax import lax
from jax.experimental import pallas as pl
from jax.experimental.pallas import tpu as pltpu
from jax.experimental.pallas import tpu_sc as plsc

_NUM_EMB = 8192
_EMB_DIM = 256
_N_ROWS = 16384

_BI = 512
_NI = _N_ROWS // _BI

_NW = 32
_ROWS_PER_WORKER = _N_ROWS // _NW
_GATHER_CHUNK = 128
_N_CHUNKS = _ROWS_PER_WORKER // _GATHER_CHUNK


def _vq_body(x_ref, w_ref, g_ref, loss_ref, idx_ref, oh_ref):
    i = pl.program_id(0)

    @pl.when(i == 0)
    def _():
        loss_ref[0, 0] = 0.0

    xs = x_ref[...]
    xn = xs / jnp.maximum(
        jnp.sqrt(jnp.sum(xs * xs, axis=1, keepdims=True)), 1e-12)

    d = lax.dot_general(
        xn, w_ref[...],
        dimension_numbers=(((1,), (1,)), ((), ())),
        preferred_element_type=jnp.float32,
    )
    nchunks = _NUM_EMB // 128
    vals = [d[:, j * 128:(j + 1) * 128] for j in range(nchunks)]
    chids = [jnp.full((_BI, 128), j, jnp.int32) for j in range(nchunks)]
    while len(vals) > 1:
        nv, nc = [], []
        for a in range(0, len(vals), 2):
            keep_l = vals[a] >= vals[a + 1]
            nv.append(jnp.where(keep_l, vals[a], vals[a + 1]))
            nc.append(jnp.where(keep_l, chids[a], chids[a + 1]))
        vals, chids = nv, nc
    v_lane, c_lane = vals[0], chids[0]

    m = jnp.max(v_lane, axis=1, keepdims=True)
    lane = lax.broadcasted_iota(jnp.int32, (_BI, 128), 1)
    qual = jnp.where(v_lane == m, c_lane * 128 + lane, _NUM_EMB)
    la = jnp.min(qual, axis=1, keepdims=True)
    idx_ref[...] = la
    cols = lax.broadcasted_iota(jnp.int32, d.shape, 1)
    oh_ref[...] = (cols == la).astype(jnp.float32)

    gs = g_ref[...]
    gdot = jnp.sum(xn * gs, axis=1, keepdims=True)
    gnorm = jnp.maximum(jnp.sqrt(jnp.sum(gs * gs, axis=1, keepdims=True)),
                        1e-12)
    loss_ref[0, 0] += jnp.sum(gdot / gnorm)

    @pl.when(i == _NI - 1)
    def _():
        loss_ref[0, 0] = 1.0 - loss_ref[0, 0] / float(_N_ROWS)


_vq_call = pl.pallas_call(
    _vq_body,
    grid=(_NI,),
    in_specs=[
        pl.BlockSpec((_BI, _EMB_DIM), lambda i: (i, 0)),
        pl.BlockSpec((_NUM_EMB, _EMB_DIM), lambda i: (0, 0)),
        pl.BlockSpec((_BI, _EMB_DIM), lambda i: (i, 0)),
    ],
    out_specs=[
        pl.BlockSpec((1, 1), lambda i: (0, 0), memory_space=pltpu.SMEM),
        pl.BlockSpec((_BI, 1), lambda i: (i, 0)),
        pl.BlockSpec((_BI, _NUM_EMB), lambda i: (i, 0)),
    ],
    out_shape=[
        jax.ShapeDtypeStruct((1, 1), jnp.float32),
        jax.ShapeDtypeStruct((_N_ROWS, 1), jnp.int32),
        jax.ShapeDtypeStruct((_N_ROWS, _NUM_EMB), jnp.float32),
    ],
)


@functools.cache
def _make_sc_gather():
    @functools.partial(
        pl.kernel,
        mesh=plsc.VectorSubcoreMesh(core_axis_name="c", subcore_axis_name="s"),
        out_type=jax.ShapeDtypeStruct((_N_ROWS, _EMB_DIM), jnp.float32),
        scratch_types=[
            pltpu.VMEM((_GATHER_CHUNK,), jnp.int32),
            pltpu.VMEM((_GATHER_CHUNK, _EMB_DIM), jnp.float32),
            pltpu.SemaphoreType.DMA,
        ],
    )
    def _sc_gather(table_hbm, idx_hbm, out_hbm, idx_v, rows_v, sem):
        wid = lax.axis_index("s") * 2 + lax.axis_index("c")
        base = wid * _ROWS_PER_WORKER
        for c in range(_N_CHUNKS):
            off = base + c * _GATHER_CHUNK
            pltpu.sync_copy(idx_hbm.at[pl.ds(off, _GATHER_CHUNK)], idx_v)
            pltpu.async_copy(table_hbm.at[idx_v], rows_v, sem).wait()
            pltpu.sync_copy(rows_v, out_hbm.at[pl.ds(off, _GATHER_CHUNK)])

    return _sc_gather


def _l2norm(t):
    n = jnp.linalg.norm(t, axis=1, keepdims=True)
    return t / jnp.maximum(n, 1e-12)


def kernel(inputs, labels, W):
    flat = inputs.reshape(-1, _EMB_DIM)
    labels_i32 = labels.astype(jnp.int32)

    sc_gather = _make_sc_gather()
    lab_rows = sc_gather(W, labels_i32)

    wn = _l2norm(W)
    loss2d, idx2d, encodings = _vq_call(flat, wn, lab_rows)

    quantized = sc_gather(W, idx2d.reshape(_N_ROWS))

    return (
        loss2d.reshape(()),
        quantized.reshape(inputs.shape),
        jnp.array(1),
        encodings,
        idx2d,
    )
